# baseline (speedup 1.0000x reference)
import functools

import jax
import jax.numpy as jnp
from jax import lax
from jax.experimental import pallas as pl
from jax.experimental.pallas import tpu as pltpu
from jax.experimental.pallas import tpu_sc as plsc

VOCAB = 100000
NUM_SENSE = 3
EMB_DIM = 64

NUM_CORES = 2
NUM_SUBCORES = 16
NW = NUM_CORES * NUM_SUBCORES
LANES = 16
NVREG = EMB_DIM // LANES
NCHUNK = 8


def _sc_fused(emb_table, disamb_table, idx, ctx):
    B = idx.shape[0]
    b_per_w = B // NW
    cb = b_per_w // NCHUNK
    n3 = NUM_SENSE * cb
    mesh = plsc.VectorSubcoreMesh(core_axis_name="c", subcore_axis_name="s")

    rows_t = pltpu.VMEM((n3, EMB_DIM), jnp.float32)
    idx3_t = pltpu.VMEM((n3,), jnp.int32)

    @functools.partial(
        pl.kernel,
        mesh=mesh,
        compiler_params=pltpu.CompilerParams(
            use_tc_tiling_on_sc=False, needs_layout_passes=False
        ),
        out_type=jax.ShapeDtypeStruct((B, EMB_DIM), jnp.float32),
        scratch_types=[
            pltpu.VMEM((b_per_w,), jnp.int32),
            idx3_t, idx3_t,
            rows_t, rows_t,
            rows_t, rows_t,
            pltpu.VMEM((cb, EMB_DIM), jnp.float32),
            pltpu.VMEM((cb, EMB_DIM), jnp.float32),
            pltpu.SemaphoreType.DMA, pltpu.SemaphoreType.DMA,
            pltpu.SemaphoreType.DMA, pltpu.SemaphoreType.DMA,
        ],
    )
    def k(emb_hbm, dis_hbm, idx_hbm, ctx_hbm, out_hbm,
          idx_v, idx3_a, idx3_b, er_a, er_b, dr_a, dr_b, ctx_v, out_v,
          sem_ea, sem_eb, sem_da, sem_db):
        wid = lax.axis_index("s") * NUM_CORES + lax.axis_index("c")
        base = wid * b_per_w
        pltpu.sync_copy(idx_hbm.at[pl.ds(base, b_per_w)], idx_v)

        bufs = ((idx3_a, er_a, dr_a, sem_ea, sem_da),
                (idx3_b, er_b, dr_b, sem_eb, sem_db))

        def issue(c):
            idx3, er, dr, se, sd = bufs[c % 2]

            @pl.loop(0, cb, step=LANES)
            def _(j):
                w3 = idx_v[pl.ds(c * cb + j, LANES)] * NUM_SENSE
                idx3[pl.ds(j, LANES)] = w3
                idx3[pl.ds(cb + j, LANES)] = w3 + 1
                idx3[pl.ds(2 * cb + j, LANES)] = w3 + 2

            ce = pltpu.async_copy(emb_hbm.at[idx3], er, se)
            cd = pltpu.async_copy(dis_hbm.at[idx3], dr, sd)
            return ce, cd

        inflight = [None, None]
        inflight[0] = issue(0)
        for c in range(NCHUNK):
            if c + 1 < NCHUNK:
                inflight[(c + 1) % 2] = issue(c + 1)
            _, er, dr, _, _ = bufs[c % 2]
            ce, cd = inflight[c % 2]
            pltpu.sync_copy(ctx_hbm.at[pl.ds(base + c * cb, cb)], ctx_v)
            cd.wait()
            ce.wait()

            @pl.loop(0, cb)
            def _(j):
                cv = [ctx_v[j, pl.ds(kk * LANES, LANES)] for kk in range(NVREG)]
                ss = []
                for s in range(NUM_SENSE):
                    acc = dr[s * cb + j, pl.ds(0, LANES)] * cv[0]
                    for kk in range(1, NVREG):
                        acc += dr[s * cb + j, pl.ds(kk * LANES, LANES)] * cv[kk]
                    ss.append(jnp.sum(acc))
                m = jnp.maximum(ss[0], jnp.maximum(ss[1], ss[2]))
                ev = [jnp.exp(lax.broadcast(ss[s] - m, (LANES,)))
                      for s in range(NUM_SENSE)]
                den = ev[0] + ev[1] + ev[2]
                for kk in range(NVREG):
                    sl = pl.ds(kk * LANES, LANES)
                    num = ev[0] * er[j, sl]
                    num += ev[1] * er[cb + j, sl]
                    num += ev[2] * er[2 * cb + j, sl]
                    out_v[j, sl] = num / den

            pltpu.sync_copy(out_v, out_hbm.at[pl.ds(base + c * cb, cb)])

    return k(emb_table, disamb_table, idx, ctx)


def kernel(word_ids, ctx, emb_table, disamb_table):
    idx = word_ids.astype(jnp.int32)
    return _sc_fused(emb_table, disamb_table, idx, ctx)


# speedup vs baseline: 1.0078x; 1.0078x over previous
import functools

import jax
import jax.numpy as jnp
from jax import lax
from jax.experimental import pallas as pl
from jax.experimental.pallas import tpu as pltpu
from jax.experimental.pallas import tpu_sc as plsc

VOCAB = 100000
NUM_SENSE = 3
EMB_DIM = 64

NUM_CORES = 2
NUM_SUBCORES = 16
NW = NUM_CORES * NUM_SUBCORES
LANES = 16
NVREG = EMB_DIM // LANES
NCHUNK = 4


def _sc_fused(emb_table, disamb_table, idx, ctx):
    B = idx.shape[0]
    b_per_w = B // NW
    cb = b_per_w // NCHUNK
    n3 = NUM_SENSE * cb
    mesh = plsc.VectorSubcoreMesh(core_axis_name="c", subcore_axis_name="s")

    rows_t = pltpu.VMEM((n3, EMB_DIM), jnp.float32)
    idx3_t = pltpu.VMEM((n3,), jnp.int32)

    @functools.partial(
        pl.kernel,
        mesh=mesh,
        compiler_params=pltpu.CompilerParams(
            use_tc_tiling_on_sc=False, needs_layout_passes=False
        ),
        out_type=jax.ShapeDtypeStruct((B, EMB_DIM), jnp.float32),
        scratch_types=[
            pltpu.VMEM((b_per_w,), jnp.int32),
            idx3_t, idx3_t,
            rows_t, rows_t,
            rows_t, rows_t,
            pltpu.VMEM((cb, EMB_DIM), jnp.float32),
            pltpu.VMEM((cb, EMB_DIM), jnp.float32),
            pltpu.SemaphoreType.DMA, pltpu.SemaphoreType.DMA,
            pltpu.SemaphoreType.DMA, pltpu.SemaphoreType.DMA,
        ],
    )
    def k(emb_hbm, dis_hbm, idx_hbm, ctx_hbm, out_hbm,
          idx_v, idx3_a, idx3_b, er_a, er_b, dr_a, dr_b, ctx_v, out_v,
          sem_ea, sem_eb, sem_da, sem_db):
        wid = lax.axis_index("s") * NUM_CORES + lax.axis_index("c")
        base = wid * b_per_w
        pltpu.sync_copy(idx_hbm.at[pl.ds(base, b_per_w)], idx_v)

        bufs = ((idx3_a, er_a, dr_a, sem_ea, sem_da),
                (idx3_b, er_b, dr_b, sem_eb, sem_db))

        def issue(c):
            idx3, er, dr, se, sd = bufs[c % 2]

            @pl.loop(0, cb, step=LANES)
            def _(j):
                w3 = idx_v[pl.ds(c * cb + j, LANES)] * NUM_SENSE
                idx3[pl.ds(j, LANES)] = w3
                idx3[pl.ds(cb + j, LANES)] = w3 + 1
                idx3[pl.ds(2 * cb + j, LANES)] = w3 + 2

            ce = pltpu.async_copy(emb_hbm.at[idx3], er, se)
            cd = pltpu.async_copy(dis_hbm.at[idx3], dr, sd)
            return ce, cd

        inflight = [None, None]
        inflight[0] = issue(0)
        for c in range(NCHUNK):
            if c + 1 < NCHUNK:
                inflight[(c + 1) % 2] = issue(c + 1)
            _, er, dr, _, _ = bufs[c % 2]
            ce, cd = inflight[c % 2]
            pltpu.sync_copy(ctx_hbm.at[pl.ds(base + c * cb, cb)], ctx_v)
            cd.wait()
            ce.wait()

            @pl.loop(0, cb)
            def _(j):
                cv = [ctx_v[j, pl.ds(kk * LANES, LANES)] for kk in range(NVREG)]
                ss = []
                for s in range(NUM_SENSE):
                    acc = dr[s * cb + j, pl.ds(0, LANES)] * cv[0]
                    for kk in range(1, NVREG):
                        acc += dr[s * cb + j, pl.ds(kk * LANES, LANES)] * cv[kk]
                    ss.append(jnp.sum(acc))
                m = jnp.maximum(ss[0], jnp.maximum(ss[1], ss[2]))
                ev = [jnp.exp(lax.broadcast(ss[s] - m, (LANES,)))
                      for s in range(NUM_SENSE)]
                den = ev[0] + ev[1] + ev[2]
                for kk in range(NVREG):
                    sl = pl.ds(kk * LANES, LANES)
                    num = ev[0] * er[j, sl]
                    num += ev[1] * er[cb + j, sl]
                    num += ev[2] * er[2 * cb + j, sl]
                    out_v[j, sl] = num / den

            pltpu.sync_copy(out_v, out_hbm.at[pl.ds(base + c * cb, cb)])

    return k(emb_table, disamb_table, idx, ctx)


def kernel(word_ids, ctx, emb_table, disamb_table):
    idx = word_ids.astype(jnp.int32)
    return _sc_fused(emb_table, disamb_table, idx, ctx)
